# Initial kernel scaffold; baseline (speedup 1.0000x reference)
import jax
import jax.numpy as jnp
from jax.experimental import pallas as pl


def kernel(node_features, actions, edge_index, sub_graphs, sep_subgraphs, Wg1a, Wg1b, Wg2a, Wg2b, gamma, beta, W1a, b1a, W2a, b2a, W3a, b3a, W1b, b1b, W2b, b2b, W3b, b3b):
    raise NotImplementedError("write your pallas kernel here")


# speedup vs baseline: 1.1670x; 1.1670x over previous
import functools

import jax
import jax.numpy as jnp
from jax.experimental import pallas as pl
from jax.experimental.pallas import tpu as pltpu

_N = 10000
_D = 128
_E = 160000
_S = 16
_H = 1024

_INTERPRET = False


def _lrelu(x):
    return jnp.where(x >= 0, x, 0.01 * x)


def _h_body(x_ref, a_ref, deg_ref, w_ref, out_ref):
    deg = jnp.maximum(deg_ref[...], 1.0)
    x2 = x_ref[...] + a_ref[...] / deg
    out_ref[...] = _lrelu(jnp.dot(x2, w_ref[...], preferred_element_type=jnp.float32))


def _h_matmul(x, a1, deg1, w1ab):
    bm = 2000
    grid = (_N // bm,)
    return pl.pallas_call(
        _h_body,
        grid=grid,
        in_specs=[
            pl.BlockSpec((bm, _D), lambda i: (i, 0)),
            pl.BlockSpec((bm, _D), lambda i: (i, 0)),
            pl.BlockSpec((bm, 1), lambda i: (i, 0)),
            pl.BlockSpec((_D, 2 * _D), lambda i: (0, 0)),
        ],
        out_specs=pl.BlockSpec((bm, 2 * _D), lambda i: (i, 0)),
        out_shape=jax.ShapeDtypeStruct((_N, 2 * _D), jnp.float32),
        interpret=_INTERPRET,
    )(x, a1, deg1, w1ab)


def _mlp_body(sgz_ref, agg_ref, deg_ref, wblk_ref,
              w1a_ref, b1a_ref, w2a_ref, b2a_ref, w3a_ref, b3a_ref,
              w1b_ref, b1b_ref, w2b_ref, b2b_ref, w3b_ref, b3b_ref,
              out_ref):
    deg = jnp.maximum(deg_ref[...], 1.0)
    pre = sgz_ref[...] + agg_ref[...] / deg
    g = jnp.dot(pre, wblk_ref[...], preferred_element_type=jnp.float32)
    lr = _lrelu(_lrelu(g))
    gb = lr.shape[0] // _S
    pooled = jnp.mean(lr.reshape(gb, _S, 2 * _D), axis=1)
    p1 = pooled[:, :_D]
    p2 = pooled[:, _D:]

    def value(p, w1, b1, w2, b2, w3, b3):
        x = _lrelu(jnp.dot(p, w1[...], preferred_element_type=jnp.float32) + b1[...])
        x = _lrelu(jnp.dot(x, w2[...], preferred_element_type=jnp.float32) + b2[...])
        return jnp.dot(x, w3[...], preferred_element_type=jnp.float32) + b3[...]

    q1 = value(p1, w1a_ref, b1a_ref, w2a_ref, b2a_ref, w3a_ref, b3a_ref)
    q2 = value(p2, w1b_ref, b1b_ref, w2b_ref, b2b_ref, w3b_ref, b3b_ref)
    out_ref[...] = jnp.concatenate([q1, q2], axis=-1)


def _mlp(sgz, agg, deg2, wblk, wt):
    bm = 6400
    gb = bm // _S
    grid = (_E // bm,)
    full = lambda shape: pl.BlockSpec(shape, lambda i: tuple(0 for _ in shape))
    return pl.pallas_call(
        _mlp_body,
        grid=grid,
        in_specs=[
            pl.BlockSpec((bm, 2 * _D), lambda i: (i, 0)),
            pl.BlockSpec((bm, 2 * _D), lambda i: (i, 0)),
            pl.BlockSpec((bm, 1), lambda i: (i, 0)),
            full((2 * _D, 2 * _D)),
            full((_D, _H)), full((1, _H)), full((_H, _H)), full((1, _H)),
            full((_H, 1)), full((1, 1)),
            full((_D, _H)), full((1, _H)), full((_H, _H)), full((1, _H)),
            full((_H, 1)), full((1, 1)),
        ],
        out_specs=pl.BlockSpec((gb, 2), lambda i: (i, 0)),
        out_shape=jax.ShapeDtypeStruct((_N, 2), jnp.float32),
        interpret=_INTERPRET,
    )(sgz, agg, deg2, wblk, *wt)


def kernel(node_features, actions, edge_index, sub_graphs, sep_subgraphs,
           Wg1a, Wg1b, Wg2a, Wg2b, gamma, beta,
           W1a, b1a, W2a, b2a, W3a, b3a,
           W1b, b1b, W2b, b2b, W3b, b3b):
    x = node_features
    src = edge_index[0]
    dst = edge_index[1]
    src2 = jnp.concatenate([src, dst])
    dst2 = jnp.concatenate([dst, src])

    deg1 = jax.ops.segment_sum(jnp.ones((2 * _E,), jnp.float32), dst2,
                               num_segments=_N)
    a1 = jax.ops.segment_sum(jnp.take(x, src2, axis=0), dst2, num_segments=_N)

    w1ab = jnp.concatenate([Wg1a, Wg1b], axis=1)
    h = _h_matmul(x, a1, deg1[:, None], w1ab)

    ef = _lrelu((jnp.take(h, src, axis=0) + jnp.take(h, dst, axis=0))
                * actions[:, None])

    mu = jnp.mean(ef, axis=0)
    var = jnp.var(ef, axis=0)
    g2 = jnp.concatenate([gamma, gamma])
    be2 = jnp.concatenate([beta, beta])
    a_bn = g2 / jnp.sqrt(var + 1e-5)
    b_bn = be2 - mu * a_bn

    efn = ef * a_bn + b_bn

    wblk = jnp.zeros((2 * _D, 2 * _D), jnp.float32)
    wblk = wblk.at[:_D, :_D].set(Wg2a).at[_D:, _D:].set(Wg2b)

    se = sep_subgraphs.reshape(-1, 2)
    s_se = jnp.concatenate([se[:, 0], se[:, 1]])
    d_se = jnp.concatenate([se[:, 1], se[:, 0]])
    idx2 = jnp.take(sub_graphs, s_se)
    deg2 = jax.ops.segment_sum(jnp.ones((_E,), jnp.float32), d_se,
                               num_segments=_E)
    agg = jax.ops.segment_sum(jnp.take(efn, idx2, axis=0), d_se,
                              num_segments=_E)
    sg = jnp.take(efn, sub_graphs, axis=0)

    wt = (W1a, b1a[None], W2a, b2a[None], W3a, b3a[None],
          W1b, b1b[None], W2b, b2b[None], W3b, b3b[None])
    return _mlp(sg, agg, deg2[:, None], wblk, wt)


# speedup vs baseline: 1.6339x; 1.4001x over previous
import functools

import jax
import jax.numpy as jnp
from jax import lax
from jax.experimental import pallas as pl
from jax.experimental.pallas import tpu as pltpu
from jax.experimental.pallas import tpu_sc as plsc

_N = 10000
_D = 128
_E = 160000
_S = 16
_H = 1024

_INTERPRET = False

_MESH = plsc.VectorSubcoreMesh(core_axis_name="c", subcore_axis_name="s")
_NC = 2
_NS = 16


def _lrelu(x):
    return jnp.where(x >= 0, x, 0.01 * x)


_A_CH = 256
_NPAD = 10240


_A_ROWS = 5632
_A_HALF = _N // 2


def _gcn1_agg_sc(x, src, dst, zeros_nd):
    per_tile = _E // _NS

    @functools.partial(
        pl.kernel,
        out_type=jax.ShapeDtypeStruct((_NC, _A_ROWS, _D), jnp.float32),
        mesh=_MESH,
        scratch_types=dict(
            acc=pltpu.VMEM_SHARED((_A_ROWS, _D), jnp.float32),
            gidx=pltpu.VMEM((_A_CH,), jnp.int32),
            sidx=pltpu.VMEM((_A_CH,), jnp.int32),
            rows=pltpu.VMEM((_A_CH, _D), jnp.float32),
            gidx_t=pltpu.VMEM((16,), jnp.int32),
            sidx_t=pltpu.VMEM((16,), jnp.int32),
            rows_t=pltpu.VMEM((16, _D), jnp.float32),
            sem=pltpu.SemaphoreType.DMA,
        ),
    )
    def k(x_hbm, src_hbm, dst_hbm, z_nd, a1_out, *, acc,
          gidx, sidx, rows, gidx_t, sidx_t, rows_t, sem):
        core = lax.axis_index("c")
        sid = lax.axis_index("s")
        stripe = _A_ROWS // _NS
        r0 = sid * stripe
        pltpu.sync_copy(z_nd.at[pl.ds(r0, stripe)], acc.at[pl.ds(r0, stripe)])
        plsc.subcore_barrier()
        base = sid * per_tile
        nb = core * _A_HALF

        def clamp(buf, n):
            @pl.loop(0, n // 16)
            def _(i):
                iv = buf[pl.ds(i * 16, 16)]
                lv = iv - nb
                m = (lv >= 0) & (lv < _A_HALF)
                buf[pl.ds(i * 16, 16)] = jnp.where(m, lv, _A_HALF)

        def chunk(off, gref, sref, gb, sb, rb, n):
            pltpu.sync_copy(gref.at[pl.ds(off, n)], gb)
            pltpu.sync_copy(sref.at[pl.ds(off, n)], sb)
            clamp(sb, n)
            pltpu.async_copy(x_hbm.at[gb], rb, sem).wait()
            pltpu.sync_copy(rb, acc.at[sb], add=True)

        nfull = per_tile // _A_CH
        tail = per_tile - nfull * _A_CH

        def direction(gref, sref):
            @pl.loop(0, nfull)
            def _(c):
                chunk(base + c * _A_CH, gref, sref, gidx, sidx, rows, _A_CH)
            if tail:
                chunk(base + nfull * _A_CH, gref, sref, gidx_t, sidx_t,
                      rows_t, tail)

        direction(src_hbm, dst_hbm)
        direction(dst_hbm, src_hbm)

        plsc.subcore_barrier()
        pltpu.sync_copy(acc.at[pl.ds(r0, stripe)],
                        a1_out.at[core, pl.ds(r0, stripe)])

    return k(x, src, dst, zeros_nd)


def _h_body(x_ref, a_ref, deg_ref, w_ref, out_ref):
    deg = jnp.maximum(deg_ref[...], 1.0)
    x2 = x_ref[...] + a_ref[...] / deg
    out_ref[...] = _lrelu(jnp.dot(x2, w_ref[...], preferred_element_type=jnp.float32))


def _h_matmul(x, a1, deg1, w1ab):
    bm = 2000
    grid = (_N // bm,)
    return pl.pallas_call(
        _h_body,
        grid=grid,
        in_specs=[
            pl.BlockSpec((bm, _D), lambda i: (i, 0)),
            pl.BlockSpec((bm, _D), lambda i: (i, 0)),
            pl.BlockSpec((bm, 1), lambda i: (i, 0)),
            pl.BlockSpec((_D, 2 * _D), lambda i: (0, 0)),
        ],
        out_specs=pl.BlockSpec((bm, 2 * _D), lambda i: (i, 0)),
        out_shape=jax.ShapeDtypeStruct((_N, 2 * _D), jnp.float32),
        interpret=_INTERPRET,
    )(x, a1, deg1, w1ab)


def _mlp_body(sgz_ref, agg_ref, deg_ref, wblk_ref,
              w1a_ref, b1a_ref, w2a_ref, b2a_ref, w3a_ref, b3a_ref,
              w1b_ref, b1b_ref, w2b_ref, b2b_ref, w3b_ref, b3b_ref,
              out_ref):
    deg = jnp.maximum(deg_ref[...], 1.0)
    pre = sgz_ref[...] + agg_ref[...] / deg
    g = jnp.dot(pre, wblk_ref[...], preferred_element_type=jnp.float32)
    lr = _lrelu(_lrelu(g))
    gb = lr.shape[0] // _S
    pooled = jnp.mean(lr.reshape(gb, _S, 2 * _D), axis=1)
    p1 = pooled[:, :_D]
    p2 = pooled[:, _D:]

    def value(p, w1, b1, w2, b2, w3, b3):
        x = _lrelu(jnp.dot(p, w1[...], preferred_element_type=jnp.float32) + b1[...])
        x = _lrelu(jnp.dot(x, w2[...], preferred_element_type=jnp.float32) + b2[...])
        return jnp.dot(x, w3[...], preferred_element_type=jnp.float32) + b3[...]

    q1 = value(p1, w1a_ref, b1a_ref, w2a_ref, b2a_ref, w3a_ref, b3a_ref)
    q2 = value(p2, w1b_ref, b1b_ref, w2b_ref, b2b_ref, w3b_ref, b3b_ref)
    out_ref[...] = jnp.concatenate([q1, q2], axis=-1)


def _mlp(sgz, agg, deg2, wblk, wt):
    bm = 6400
    gb = bm // _S
    grid = (_E // bm,)
    full = lambda shape: pl.BlockSpec(shape, lambda i: tuple(0 for _ in shape))
    return pl.pallas_call(
        _mlp_body,
        grid=grid,
        in_specs=[
            pl.BlockSpec((bm, 2 * _D), lambda i: (i, 0)),
            pl.BlockSpec((bm, 2 * _D), lambda i: (i, 0)),
            pl.BlockSpec((bm, 1), lambda i: (i, 0)),
            full((2 * _D, 2 * _D)),
            full((_D, _H)), full((1, _H)), full((_H, _H)), full((1, _H)),
            full((_H, 1)), full((1, 1)),
            full((_D, _H)), full((1, _H)), full((_H, _H)), full((1, _H)),
            full((_H, 1)), full((1, 1)),
        ],
        out_specs=pl.BlockSpec((gb, 2), lambda i: (i, 0)),
        out_shape=jax.ShapeDtypeStruct((_N, 2), jnp.float32),
        interpret=_INTERPRET,
    )(sgz, agg, deg2, wblk, *wt)


def kernel(node_features, actions, edge_index, sub_graphs, sep_subgraphs,
           Wg1a, Wg1b, Wg2a, Wg2b, gamma, beta,
           W1a, b1a, W2a, b2a, W3a, b3a,
           W1b, b1b, W2b, b2b, W3b, b3b):
    x = node_features
    src = edge_index[0]
    dst = edge_index[1]
    src2 = jnp.concatenate([src, dst])
    dst2 = jnp.concatenate([dst, src])

    a1p = _gcn1_agg_sc(x, src, dst, jnp.zeros((_A_ROWS, _D), jnp.float32))
    a1 = jnp.concatenate([a1p[0, :_A_HALF], a1p[1, :_A_HALF]], axis=0)
    dst2 = jnp.concatenate([dst, src])
    deg1 = jax.ops.segment_sum(jnp.ones((2 * _E,), jnp.float32), dst2,
                               num_segments=_N)

    w1ab = jnp.concatenate([Wg1a, Wg1b], axis=1)
    h = _h_matmul(x, a1, deg1[:, None], w1ab)

    ef = _lrelu((jnp.take(h, src, axis=0) + jnp.take(h, dst, axis=0))
                * actions[:, None])

    mu = jnp.mean(ef, axis=0)
    var = jnp.var(ef, axis=0)
    g2 = jnp.concatenate([gamma, gamma])
    be2 = jnp.concatenate([beta, beta])
    a_bn = g2 / jnp.sqrt(var + 1e-5)
    b_bn = be2 - mu * a_bn

    efn = ef * a_bn + b_bn

    wblk = jnp.zeros((2 * _D, 2 * _D), jnp.float32)
    wblk = wblk.at[:_D, :_D].set(Wg2a).at[_D:, _D:].set(Wg2b)

    se = sep_subgraphs.reshape(-1, 2)
    s_se = jnp.concatenate([se[:, 0], se[:, 1]])
    d_se = jnp.concatenate([se[:, 1], se[:, 0]])
    idx2 = jnp.take(sub_graphs, s_se)
    deg2 = jax.ops.segment_sum(jnp.ones((_E,), jnp.float32), d_se,
                               num_segments=_E)
    agg = jax.ops.segment_sum(jnp.take(efn, idx2, axis=0), d_se,
                              num_segments=_E)
    sg = jnp.take(efn, sub_graphs, axis=0)

    wt = (W1a, b1a[None], W2a, b2a[None], W3a, b3a[None],
          W1b, b1b[None], W2b, b2b[None], W3b, b3b[None])
    return _mlp(sg, agg, deg2[:, None], wblk, wt)
